# baseline (speedup 1.0000x reference)
import jax
import jax.numpy as jnp
from jax.experimental import pallas as pl
from jax.experimental.pallas import tpu as pltpu

_D, _HW = 256, 25
_TRI = _D * (_D + 1) // 2


def _bdc_kernel(fc_ref, temp_ref, out_ref, scr, pk, sems):
    i = pl.program_id(0)
    nb = pl.num_programs(0)
    slot = jax.lax.rem(i, 2)

    def _pk_copy(s, row):
        return pltpu.make_async_copy(
            pk.at[s], out_ref.at[pl.ds(row, 1), :], sems.at[s]
        )

    @pl.when(i >= 3)
    def _():
        _pk_copy(slot, 0).wait()

    fc = fc_ref[0]
    ft = fc.T
    ra = jnp.abs(ft)
    rn = -ra
    rs2 = jnp.where(ft < 0.0, -2.0, 2.0)
    acc = jnp.zeros((_D, _D), jnp.float32)
    for p in range(_HW):
        c = fc[:, p : p + 1]
        t = jnp.minimum(jnp.maximum(c, rn[p : p + 1, :]), ra[p : p + 1, :])
        acc = acc + rs2[p : p + 1, :] * t
    row_mean = jnp.sum(acc, axis=1, keepdims=True) * (1.0 / _D)
    col_mean = jnp.sum(acc, axis=0, keepdims=True) * (1.0 / _D)
    scale = 0.5 * jnp.exp(temp_ref[...])
    scr[slot] = (acc - row_mean - col_mean) * scale

    def _pack(src_slot, dst_slot):
        for r in range(_D):
            off = r * _D - (r * (r - 1)) // 2
            pk[dst_slot, 0, pl.ds(off, _D - r)] = scr[src_slot, r, r:]

    _pack(1 - slot, slot)

    @pl.when(i >= 1)
    def _():
        _pk_copy(slot, i - 1).start()

    @pl.when(i == nb - 1)
    def _():
        _pk_copy(1 - slot, 0).wait()
        _pack(slot, 1 - slot)
        _pk_copy(1 - slot, nb - 1).start()
        _pk_copy(slot, 0).wait()
        _pk_copy(1 - slot, 0).wait()


def kernel(feat_map, temperature):
    b, d, h, w = feat_map.shape
    fc = feat_map.reshape(b, d, h * w)
    packed = pl.pallas_call(
        _bdc_kernel,
        grid=(b,),
        in_specs=[
            pl.BlockSpec((1, d, h * w), lambda i: (i, 0, 0)),
            pl.BlockSpec((1, 1), lambda i: (0, 0)),
        ],
        out_specs=pl.BlockSpec(memory_space=pl.ANY),
        out_shape=jax.ShapeDtypeStruct((b, _TRI), jnp.float32),
        scratch_shapes=[
            pltpu.VMEM((2, _D, _D), jnp.float32),
            pltpu.VMEM((2, 1, _TRI), jnp.float32),
            pltpu.SemaphoreType.DMA((2,)),
        ],
        compiler_params=pltpu.CompilerParams(
            dimension_semantics=("arbitrary",),
        ),
        name="bdc_pool",
    )(fc, temperature)
    return packed


# speedup vs baseline: 1.0451x; 1.0451x over previous
import jax
import jax.numpy as jnp
from jax.experimental import pallas as pl
from jax.experimental.pallas import tpu as pltpu

_D, _HW = 256, 25
_TRI = _D * (_D + 1) // 2


def _bdc_kernel(fc_ref, temp_ref, out_ref, scr, pk, sems):
    i = pl.program_id(0)
    nb = pl.num_programs(0)
    slot = jax.lax.rem(i, 2)

    def _pk_copy(s, row):
        return pltpu.make_async_copy(
            pk.at[s], out_ref.at[pl.ds(row, 1), :], sems.at[s]
        )

    @pl.when(i >= 3)
    def _():
        _pk_copy(slot, 0).wait()

    def _compute_and_pack(this_slot):
        fc = fc_ref[0]
        ft = fc.T
        ra = jnp.abs(ft)
        rn = -ra
        rs2 = jnp.where(ft < 0.0, -2.0, 2.0)
        acc = jnp.zeros((_D, _D), jnp.float32)
        for p in range(_HW):
            c = fc[:, p : p + 1]
            t = jnp.minimum(jnp.maximum(c, rn[p : p + 1, :]), ra[p : p + 1, :])
            acc = acc + rs2[p : p + 1, :] * t
        row_mean = jnp.sum(acc, axis=1, keepdims=True) * (1.0 / _D)
        col_mean = jnp.sum(acc, axis=0, keepdims=True) * (1.0 / _D)
        scale = 0.5 * jnp.exp(temp_ref[...])
        scr[this_slot] = (acc - row_mean - col_mean) * scale
        _pack(1 - this_slot, this_slot)

    def _pack(src_slot, dst_slot):
        for r in range(_D):
            off = r * _D - (r * (r - 1)) // 2
            pk[dst_slot, 0, pl.ds(off, _D - r)] = scr[src_slot, r, r:]

    @pl.when(slot == 0)
    def _():
        _compute_and_pack(0)

    @pl.when(slot == 1)
    def _():
        _compute_and_pack(1)

    @pl.when(i >= 1)
    def _():
        _pk_copy(slot, i - 1).start()

    @pl.when(i == nb - 1)
    def _():
        _pk_copy(1 - slot, 0).wait()
        _pack(slot, 1 - slot)
        _pk_copy(1 - slot, nb - 1).start()
        _pk_copy(slot, 0).wait()
        _pk_copy(1 - slot, 0).wait()


def kernel(feat_map, temperature):
    b, d, h, w = feat_map.shape
    fc = feat_map.reshape(b, d, h * w)
    packed = pl.pallas_call(
        _bdc_kernel,
        grid=(b,),
        in_specs=[
            pl.BlockSpec((1, d, h * w), lambda i: (i, 0, 0)),
            pl.BlockSpec((1, 1), lambda i: (0, 0)),
        ],
        out_specs=pl.BlockSpec(memory_space=pl.ANY),
        out_shape=jax.ShapeDtypeStruct((b, _TRI), jnp.float32),
        scratch_shapes=[
            pltpu.VMEM((2, _D, _D), jnp.float32),
            pltpu.VMEM((2, 1, _TRI), jnp.float32),
            pltpu.SemaphoreType.DMA((2,)),
        ],
        compiler_params=pltpu.CompilerParams(
            dimension_semantics=("arbitrary",),
        ),
        name="bdc_pool",
    )(fc, temperature)
    return packed
